# baseline (speedup 1.0000x reference)
import jax
import jax.numpy as jnp
from jax.experimental import pallas as pl
from jax.experimental.pallas import tpu as pltpu

_MB = 1024 * 1024


def _matmul_bias_kernel(x_ref, w_ref, b_ref, o_ref):
    acc = jnp.dot(x_ref[...], w_ref[...], preferred_element_type=jnp.float32)
    o_ref[...] = (acc + b_ref[...].astype(jnp.float32)).astype(o_ref.dtype)


def _pick_tm(n):
    for tm in (512, 256, 128, 64, 32, 16, 8):
        if n % tm == 0 and (n // tm) % 2 == 0:
            return tm
    for tm in (512, 256, 128, 64, 32, 16, 8):
        if n % tm == 0:
            return tm
    return None


def _linear2d(x2, w, b2):
    n, f_in = x2.shape
    f_out = w.shape[1]
    dtype = x2.dtype
    itemsize = jnp.dtype(dtype).itemsize

    tm = _pick_tm(n)
    if tm is None:
        tm = min(n, 1024)
    grid = (pl.cdiv(n, tm),)

    cost = pl.CostEstimate(
        flops=2 * n * f_in * f_out,
        transcendentals=0,
        bytes_accessed=itemsize * (n * f_in + f_in * f_out + f_out + n * f_out),
    )
    vmem_limit = min(
        2 * (f_in * f_out + f_out) * itemsize + 5 * tm * (f_in + f_out) * itemsize
        + 4 * _MB,
        56 * _MB,
    )

    return pl.pallas_call(
        _matmul_bias_kernel,
        out_shape=jax.ShapeDtypeStruct((n, f_out), dtype),
        grid=grid,
        in_specs=[
            pl.BlockSpec((tm, f_in), lambda i: (i, 0)),
            pl.BlockSpec((f_in, f_out), lambda i: (0, 0)),
            pl.BlockSpec((1, f_out), lambda i: (0, 0)),
        ],
        out_specs=pl.BlockSpec((tm, f_out), lambda i: (i, 0)),
        compiler_params=pltpu.CompilerParams(
            dimension_semantics=("parallel",),
            vmem_limit_bytes=int(vmem_limit),
        ),
        cost_estimate=cost,
    )(x2, w, b2)


def kernel(x, w, b):
    f_out = w.shape[1]
    b2 = b.reshape(1, f_out)
    if x.ndim <= 2:
        x2 = x.reshape(1, -1) if x.ndim == 1 else x
        y = _linear2d(x2, w, b2)
        return y.reshape(-1) if x.ndim == 1 else y
    x2 = x.reshape(-1, x.shape[-1])
    y = _linear2d(x2, w, b2)
    return y.reshape(-1, x.shape[1], f_out)


# speedup vs baseline: 1.2349x; 1.2349x over previous
import jax
import jax.numpy as jnp
from jax.experimental import pallas as pl
from jax.experimental.pallas import tpu as pltpu

_MB = 1024 * 1024


def _matmul_bias_kernel(x_ref, w_ref, b_ref, o_ref):
    acc = jnp.dot(x_ref[...], w_ref[...], preferred_element_type=jnp.float32)
    o_ref[...] = (acc + b_ref[...].astype(jnp.float32)).astype(o_ref.dtype)


def _pick_tm(n):
    for tm in (2048, 1024, 512, 256, 128, 64, 32, 16, 8):
        if n % tm == 0 and (n // tm) % 2 == 0:
            return tm
    for tm in (2048, 1024, 512, 256, 128, 64, 32, 16, 8):
        if n % tm == 0:
            return tm
    return None


def _linear2d(x2, w, b2):
    n, f_in = x2.shape
    f_out = w.shape[1]
    dtype = x2.dtype
    itemsize = jnp.dtype(dtype).itemsize

    tm = _pick_tm(n)
    if tm is None:
        tm = min(n, 1024)
    grid = (pl.cdiv(n, tm),)

    cost = pl.CostEstimate(
        flops=2 * n * f_in * f_out,
        transcendentals=0,
        bytes_accessed=itemsize * (n * f_in + f_in * f_out + f_out + n * f_out),
    )
    vmem_limit = min(
        2 * (f_in * f_out + f_out) * itemsize + 5 * tm * (f_in + f_out) * itemsize
        + 4 * _MB,
        56 * _MB,
    )

    return pl.pallas_call(
        _matmul_bias_kernel,
        out_shape=jax.ShapeDtypeStruct((n, f_out), dtype),
        grid=grid,
        in_specs=[
            pl.BlockSpec((tm, f_in), lambda i: (i, 0)),
            pl.BlockSpec((f_in, f_out), lambda i: (0, 0)),
            pl.BlockSpec((1, f_out), lambda i: (0, 0)),
        ],
        out_specs=pl.BlockSpec((tm, f_out), lambda i: (i, 0)),
        compiler_params=pltpu.CompilerParams(
            dimension_semantics=("parallel",),
            vmem_limit_bytes=int(vmem_limit),
        ),
        cost_estimate=cost,
    )(x2, w, b2)


def kernel(x, w, b):
    f_out = w.shape[1]
    b2 = b.reshape(1, f_out)
    if x.ndim <= 2:
        x2 = x.reshape(1, -1) if x.ndim == 1 else x
        y = _linear2d(x2, w, b2)
        return y.reshape(-1) if x.ndim == 1 else y
    x2 = x.reshape(-1, x.shape[-1])
    y = _linear2d(x2, w, b2)
    return y.reshape(-1, x.shape[1], f_out)
